# Initial kernel scaffold; baseline (speedup 1.0000x reference)
import functools
import math

import jax
import jax.numpy as jnp
from jax import lax
from jax.experimental import pallas as pl
from jax.experimental.pallas import tpu as pltpu
from jax.experimental.pallas import tpu_sc as plsc

HEADS = 4
D_HEAD = 64
HIDDEN = 256
HAN_OUT = 64
HAN_HEADS = 4
HAN_D = 16

NC = 2
NS = 16
LANES = 16
NW = NC * NS

CH = 128
NT = 10240
RPT = NT // NS

BR = 400


def _t1_body(xr_ref, x_ref_, wproj, bproj, as_m, ad_m, wlin, blin, wq, bq, wk, bk,
             wv, bv, bq_rel, bv_rel, qscale, mh,
             h_o, qs_o, kt_o, v2_o, xrh_o, asrc_o, adst_o, nq_o, nk_o):
    xr = xr_ref[...]
    xp = jnp.dot(xr, wproj[...], preferred_element_type=jnp.float32) + bproj[...]
    xrh_o[...] = xp
    asrc_o[...] = jnp.dot(xp, as_m[...], preferred_element_type=jnp.float32)
    adst_o[...] = jnp.dot(xp, ad_m[...], preferred_element_type=jnp.float32)

    x = x_ref_[...]
    h = jnp.maximum(jnp.dot(x, wlin[...], preferred_element_type=jnp.float32) + blin[...], 0.0)
    h_o[...] = h
    q = jnp.dot(h, wq[...], preferred_element_type=jnp.float32) + bq[...]
    k = jnp.dot(h, wk[...], preferred_element_type=jnp.float32) + bk[...]
    v = jnp.dot(h, wv[...], preferred_element_type=jnp.float32) + bv[...]
    kt = jnp.dot(k, bq_rel[...], preferred_element_type=jnp.float32)
    vt = jnp.dot(v, bv_rel[...], preferred_element_type=jnp.float32)
    qs = q * qscale[...]
    qs_o[...] = qs
    kt_o[...] = kt
    v2_o[0] = vt[:, :128]
    v2_o[1] = vt[:, 128:]
    nq_o[...] = jnp.sqrt(jnp.dot(qs * qs, mh[...], preferred_element_type=jnp.float32))
    nk_o[...] = jnp.sqrt(jnp.dot(kt * kt, mh[...], preferred_element_type=jnp.float32))


def _t1b_body(nq_ref, nk_ref, asrc_ref, adst_ref, c_o, cr_o):
    kmax = jnp.max(nk_ref[...], axis=0, keepdims=True)
    c_o[...] = nq_ref[...] * kmax
    amax = jnp.max(asrc_ref[...], axis=0, keepdims=True)
    t = amax + adst_ref[...]
    cr_o[...] = jnp.where(t > 0, t, 0.2 * t)


def _tmid_body(sh_ref, sr_ref, rsh_o, rsr_o):
    rsh_o[...] = 1.0 / (sh_ref[0] + sh_ref[1] + 1e-16)
    rsr_o[...] = 1.0 / (sr_ref[0] + sr_ref[1] + 1e-16)


def _t2_body(lo_ref, hi_ref, orr_ref, h_ref, wout, bout, skip_ref, wf1, wf2, bfin,
             emb_o):
    og = jnp.concatenate([lo_ref[0] + lo_ref[1], hi_ref[0] + hi_ref[1]], axis=1)
    g = jax.nn.gelu(og, approximate=False)
    out = jnp.dot(g, wout[...], preferred_element_type=jnp.float32) + bout[...]
    beta = jax.nn.sigmoid(skip_ref[0, 0])
    hn = beta * out + (1.0 - beta) * h_ref[...]
    orr = jnp.maximum(orr_ref[0] + orr_ref[1], 0.0)
    emb_o[...] = (jnp.dot(hn, wf1[...], preferred_element_type=jnp.float32)
                  + jnp.dot(orr, wf2[...], preferred_element_type=jnp.float32)
                  + bfin[...])


def _full16(val):
    return jnp.full((LANES,), val, dtype=jnp.int32)


def _make_s1(e_pad, n):
    ew = e_pad // NW
    nchunk = ew // CH
    mesh = plsc.VectorSubcoreMesh(core_axis_name="c", subcore_axis_name="s")

    @functools.partial(
        pl.kernel,
        mesh=mesh,
        out_type=[
            jax.ShapeDtypeStruct((e_pad, HEADS), jnp.float32),
            jax.ShapeDtypeStruct((e_pad, HEADS), jnp.float32),
            jax.ShapeDtypeStruct((NC, NT, HEADS), jnp.float32),
            jax.ShapeDtypeStruct((NC, NT, HEADS), jnp.float32),
        ],
        scratch_types=[
            pltpu.VMEM((CH,), jnp.int32),
            pltpu.VMEM((CH,), jnp.int32),
            pltpu.VMEM((CH,), jnp.int32),
            pltpu.VMEM((CH,), jnp.int32),
            pltpu.VMEM((CH, HIDDEN), jnp.float32),
            pltpu.VMEM((CH, HIDDEN), jnp.float32),
            pltpu.VMEM((CH, HEADS), jnp.float32),
            pltpu.VMEM((CH, HEADS), jnp.float32),
            pltpu.VMEM((CH, HEADS), jnp.float32),
            pltpu.VMEM((CH, HEADS), jnp.float32),
            pltpu.VMEM((CH, HEADS), jnp.float32),
            pltpu.VMEM((CH, HEADS), jnp.float32),
            pltpu.VMEM_SHARED((NT, HEADS), jnp.float32),
            pltpu.VMEM_SHARED((NT, HEADS), jnp.float32),
        ],
    )
    def s1(src_h, dst_h, srcr_h, dstr_h, qs_h, kt_h, c_h, asrc_h, adst_h, cr_h, z4_h,
           ex_hgt_h, ex_han_h, s_hgt_h, s_han_h,
           sv, dv, srv, drv, qrows, krows, crows, asr, adr, crr, exb, exrb,
           stab, stab_r):
        cid = lax.axis_index("c")
        sid = lax.axis_index("s")
        wid = sid * NC + cid
        r0 = sid * RPT
        pltpu.sync_copy(z4_h.at[pl.ds(r0, RPT)], stab.at[pl.ds(r0, RPT)])
        pltpu.sync_copy(z4_h.at[pl.ds(r0, RPT)], stab_r.at[pl.ds(r0, RPT)])
        plsc.subcore_barrier()
        lane = lax.iota(jnp.int32, LANES)
        base_e = wid * ew

        def chunk(ci, carry):
            eb = base_e + ci * CH
            pltpu.sync_copy(src_h.at[pl.ds(eb, CH)], sv)
            pltpu.sync_copy(dst_h.at[pl.ds(eb, CH)], dv)
            pltpu.sync_copy(qs_h.at[dv], qrows)
            pltpu.sync_copy(kt_h.at[sv], krows)
            pltpu.sync_copy(c_h.at[dv], crows)
            for g in range(CH // LANES):
                lids = lane + (g * LANES)

                def dstep(d, accs):
                    cb = jnp.full((LANES,), d, dtype=jnp.int32)
                    new = []
                    for h in range(HEADS):
                        col = cb + (h * D_HEAD)
                        qv = plsc.load_gather(qrows, [lids, col])
                        kv = plsc.load_gather(krows, [lids, col])
                        new.append(accs[h] + qv * kv)
                    return tuple(new)

                zero = jnp.zeros((LANES,), jnp.float32)
                accs = lax.fori_loop(0, D_HEAD, dstep, (zero, zero, zero, zero))
                for h in range(HEADS):
                    hcol = _full16(h)
                    cv = plsc.load_gather(crows, [lids, hcol])
                    ex = jnp.exp(accs[h] - cv)
                    plsc.store_scatter(exb, [lids, hcol], ex)
            pltpu.sync_copy(exb, stab.at[dv], add=True)
            pltpu.sync_copy(exb, ex_hgt_h.at[pl.ds(eb, CH)])

            pltpu.sync_copy(srcr_h.at[pl.ds(eb, CH)], srv)
            pltpu.sync_copy(dstr_h.at[pl.ds(eb, CH)], drv)
            pltpu.sync_copy(asrc_h.at[srv], asr)
            pltpu.sync_copy(adst_h.at[drv], adr)
            pltpu.sync_copy(cr_h.at[drv], crr)
            for g in range(CH // LANES):
                lids = lane + (g * LANES)
                for h in range(HEADS):
                    hcol = _full16(h)
                    a = (plsc.load_gather(asr, [lids, hcol])
                         + plsc.load_gather(adr, [lids, hcol]))
                    a = jnp.where(a > 0, a, 0.2 * a)
                    exr = jnp.exp(a - plsc.load_gather(crr, [lids, hcol]))
                    plsc.store_scatter(exrb, [lids, hcol], exr)
            pltpu.sync_copy(exrb, stab_r.at[drv], add=True)
            pltpu.sync_copy(exrb, ex_han_h.at[pl.ds(eb, CH)])
            return carry

        lax.fori_loop(0, nchunk, chunk, 0)
        plsc.subcore_barrier()
        pltpu.sync_copy(stab.at[pl.ds(r0, RPT)], s_hgt_h.at[cid, pl.ds(r0, RPT)])
        pltpu.sync_copy(stab_r.at[pl.ds(r0, RPT)], s_han_h.at[cid, pl.ds(r0, RPT)])

    return s1


def _make_s3(e_pad, n):
    ew = e_pad // NW
    nchunk = ew // CH
    mesh = plsc.VectorSubcoreMesh(core_axis_name="c", subcore_axis_name="s")
    HALF = 128

    @functools.partial(
        pl.kernel,
        mesh=mesh,
        out_type=[
            jax.ShapeDtypeStruct((e_pad, HEADS), jnp.float32),
            jax.ShapeDtypeStruct((NC, NT, HALF), jnp.float32),
            jax.ShapeDtypeStruct((NC, NT, HALF), jnp.float32),
            jax.ShapeDtypeStruct((NC, NT, HAN_OUT), jnp.float32),
        ],
        scratch_types=[
            pltpu.VMEM((CH,), jnp.int32),
            pltpu.VMEM((CH,), jnp.int32),
            pltpu.VMEM((CH,), jnp.int32),
            pltpu.VMEM((CH,), jnp.int32),
            pltpu.VMEM((CH,), jnp.int32),
            pltpu.VMEM((CH, HEADS), jnp.float32),
            pltpu.VMEM((CH, HEADS), jnp.float32),
            pltpu.VMEM((CH, HEADS), jnp.float32),
            pltpu.VMEM((CH, HEADS), jnp.float32),
            pltpu.VMEM((CH, HEADS), jnp.float32),
            pltpu.VMEM((CH, HEADS), jnp.float32),
            pltpu.VMEM((CH, HALF), jnp.float32),
            pltpu.VMEM((CH, HALF), jnp.float32),
            pltpu.VMEM((CH, HAN_OUT), jnp.float32),
            pltpu.VMEM((CH, HAN_OUT), jnp.float32),
            pltpu.VMEM_SHARED((NT, HALF), jnp.float32),
            pltpu.VMEM_SHARED((NT, HAN_OUT), jnp.float32),
        ],
    )
    def s3(src_h, srcn_h, dst_h, srcr_h, dstr_h, ex_hgt_h, ex_han_h, rsh_h, rsr_h,
           v2_h, xr_h, z128_h, z64_h,
           al_h, olo_h, ohi_h, or_h,
           sv, svn, dv, srv, drv, exb, rsb, alb, exrb, rsrb, alrb,
           vrows, msgb, xrows, msgrb, otab, ortab):
        cid = lax.axis_index("c")
        sid = lax.axis_index("s")
        wid = sid * NC + cid
        r0 = sid * RPT
        lane = lax.iota(jnp.int32, LANES)
        base_e = wid * ew

        pltpu.sync_copy(z128_h.at[pl.ds(r0, RPT)], otab.at[pl.ds(r0, RPT)])
        pltpu.sync_copy(z64_h.at[pl.ds(r0, RPT)], ortab.at[pl.ds(r0, RPT)])
        plsc.subcore_barrier()

        def do_pass(half):
            h0, h1 = (0, 1) if half == 0 else (2, 3)
            idx_h = src_h if half == 0 else srcn_h
            idx_v = sv if half == 0 else svn

            def chunk(ci, carry):
                eb = base_e + ci * CH
                pltpu.sync_copy(idx_h.at[pl.ds(eb, CH)], idx_v)
                pltpu.sync_copy(dst_h.at[pl.ds(eb, CH)], dv)
                pltpu.sync_copy(ex_hgt_h.at[pl.ds(eb, CH)], exb)
                pltpu.sync_copy(rsh_h.at[dv], rsb)
                for g in range(CH // LANES):
                    lids = lane + (g * LANES)
                    for h in range(HEADS):
                        hcol = _full16(h)
                        al = (plsc.load_gather(exb, [lids, hcol])
                              * plsc.load_gather(rsb, [lids, hcol]))
                        plsc.store_scatter(alb, [lids, hcol], al)
                if half == 0:
                    pltpu.sync_copy(alb, al_h.at[pl.ds(eb, CH)])
                pltpu.sync_copy(v2_h.at[idx_v], vrows)

                def vbody(ei, c2):
                    av0 = plsc.load_gather(alb, [_full16(ei), _full16(h0)])
                    av1 = plsc.load_gather(alb, [_full16(ei), _full16(h1)])
                    for c8 in range(HALF // LANES):
                        av = av0 if c8 < 4 else av1
                        v16 = vrows[ei, pl.ds(c8 * LANES, LANES)]
                        msgb[ei, pl.ds(c8 * LANES, LANES)] = v16 * av
                    return c2

                lax.fori_loop(0, CH, vbody, 0)
                pltpu.sync_copy(msgb, otab.at[dv], add=True)

                if half == 0:
                    pltpu.sync_copy(srcr_h.at[pl.ds(eb, CH)], srv)
                    pltpu.sync_copy(dstr_h.at[pl.ds(eb, CH)], drv)
                    pltpu.sync_copy(ex_han_h.at[pl.ds(eb, CH)], exrb)
                    pltpu.sync_copy(rsr_h.at[drv], rsrb)
                    for g in range(CH // LANES):
                        lids = lane + (g * LANES)
                        for h in range(HEADS):
                            hcol = _full16(h)
                            al = (plsc.load_gather(exrb, [lids, hcol])
                                  * plsc.load_gather(rsrb, [lids, hcol]))
                            plsc.store_scatter(alrb, [lids, hcol], al)
                    pltpu.sync_copy(xr_h.at[srv], xrows)

                    def rbody(ei, c2):
                        for h in range(HAN_HEADS):
                            av = plsc.load_gather(alrb, [_full16(ei), _full16(h)])
                            x16 = xrows[ei, pl.ds(h * HAN_D, LANES)]
                            msgrb[ei, pl.ds(h * HAN_D, LANES)] = x16 * av
                        return c2

                    lax.fori_loop(0, CH, rbody, 0)
                    pltpu.sync_copy(msgrb, ortab.at[drv], add=True)
                return carry

            lax.fori_loop(0, nchunk, chunk, 0)

        do_pass(0)
        plsc.subcore_barrier()
        pltpu.sync_copy(otab.at[pl.ds(r0, RPT)], olo_h.at[cid, pl.ds(r0, RPT)])
        pltpu.sync_copy(ortab.at[pl.ds(r0, RPT)], or_h.at[cid, pl.ds(r0, RPT)])
        plsc.subcore_barrier()
        pltpu.sync_copy(z128_h.at[pl.ds(r0, RPT)], otab.at[pl.ds(r0, RPT)])
        plsc.subcore_barrier()
        do_pass(1)
        plsc.subcore_barrier()
        pltpu.sync_copy(otab.at[pl.ds(r0, RPT)], ohi_h.at[cid, pl.ds(r0, RPT)])

    return s3


def kernel(x, edge_index, x_ref, edge_index_ref, W_lin, b_lin, W_kqv, b_kqv,
           W_krel, W_vrel, p_rel, W_out, b_out, skip, W_proj, b_proj, att_src,
           att_dst, W_klin, b_klin, q_sem, W_fin, b_fin):
    n = x.shape[0]
    e = edge_index.shape[1]
    e_ref = edge_index_ref.shape[1]
    d_in = x.shape[1]

    wk = W_kqv[:, :HIDDEN]
    wq = W_kqv[:, HIDDEN:2 * HIDDEN]
    wv = W_kqv[:, 2 * HIDDEN:]
    bk = b_kqv[:HIDDEN]
    bq = b_kqv[HIDDEN:2 * HIDDEN]
    bv = b_kqv[2 * HIDDEN:]
    bq_rel = jax.scipy.linalg.block_diag(*[W_krel[h] for h in range(HEADS)])
    bv_rel = jax.scipy.linalg.block_diag(*[W_vrel[h] for h in range(HEADS)])
    qscale = jnp.repeat(p_rel / math.sqrt(D_HEAD), D_HEAD)[None, :]
    mh = jax.scipy.linalg.block_diag(*[jnp.ones((D_HEAD, 1), jnp.float32)
                                       for _ in range(HEADS)])
    as_m = jax.scipy.linalg.block_diag(*[att_src[h][:, None]
                                         for h in range(HAN_HEADS)])
    ad_m = jax.scipy.linalg.block_diag(*[att_dst[h][:, None]
                                         for h in range(HAN_HEADS)])

    grid = n // BR
    bw = lambda shp: pl.BlockSpec(shp, lambda *i: tuple(0 for _ in shp))
    t1 = pl.pallas_call(
        _t1_body,
        grid=(grid,),
        in_specs=[
            pl.BlockSpec((BR, d_in), lambda i: (i, 0)),
            pl.BlockSpec((BR, d_in), lambda i: (i, 0)),
            bw((d_in, HAN_OUT)), bw((1, HAN_OUT)),
            bw((HAN_OUT, HEADS)), bw((HAN_OUT, HEADS)),
            bw((d_in, HIDDEN)), bw((1, HIDDEN)),
            bw((HIDDEN, HIDDEN)), bw((1, HIDDEN)),
            bw((HIDDEN, HIDDEN)), bw((1, HIDDEN)),
            bw((HIDDEN, HIDDEN)), bw((1, HIDDEN)),
            bw((HIDDEN, HIDDEN)), bw((HIDDEN, HIDDEN)),
            bw((1, HIDDEN)), bw((HIDDEN, HEADS)),
        ],
        out_specs=[
            pl.BlockSpec((BR, HIDDEN), lambda i: (i, 0)),
            pl.BlockSpec((BR, HIDDEN), lambda i: (i, 0)),
            pl.BlockSpec((BR, HIDDEN), lambda i: (i, 0)),
            pl.BlockSpec((2, BR, 128), lambda i: (0, i, 0)),
            pl.BlockSpec((BR, HAN_OUT), lambda i: (i, 0)),
            pl.BlockSpec((BR, HEADS), lambda i: (i, 0)),
            pl.BlockSpec((BR, HEADS), lambda i: (i, 0)),
            pl.BlockSpec((BR, HEADS), lambda i: (i, 0)),
            pl.BlockSpec((BR, HEADS), lambda i: (i, 0)),
        ],
        out_shape=[
            jax.ShapeDtypeStruct((n, HIDDEN), jnp.float32),
            jax.ShapeDtypeStruct((n, HIDDEN), jnp.float32),
            jax.ShapeDtypeStruct((n, HIDDEN), jnp.float32),
            jax.ShapeDtypeStruct((2, n, 128), jnp.float32),
            jax.ShapeDtypeStruct((n, HAN_OUT), jnp.float32),
            jax.ShapeDtypeStruct((n, HEADS), jnp.float32),
            jax.ShapeDtypeStruct((n, HEADS), jnp.float32),
            jax.ShapeDtypeStruct((n, HEADS), jnp.float32),
            jax.ShapeDtypeStruct((n, HEADS), jnp.float32),
        ],
    )(x_ref, x, W_proj, b_proj[None, :], as_m, ad_m, W_lin, b_lin[None, :],
      wq, bq[None, :], wk, bk[None, :], wv, bv[None, :], bq_rel, bv_rel,
      qscale, mh)
    h, qs, kt, v2, xr, asrc, adst, nq, nk = t1

    c, cr = pl.pallas_call(
        _t1b_body,
        out_shape=[jax.ShapeDtypeStruct((n, HEADS), jnp.float32),
                   jax.ShapeDtypeStruct((n, HEADS), jnp.float32)],
    )(nq, nk, asrc, adst)

    pad8 = lambda a: jnp.pad(a, ((0, 8), (0, 0)))
    qs_p = pad8(qs)
    c_p = pad8(c)
    adst_p = pad8(adst)
    cr_p = pad8(cr)

    def pad_edges(ei, e_cnt):
        e_pad = ((e_cnt + NW * CH - 1) // (NW * CH)) * (NW * CH)
        src = jnp.pad(ei[0], (0, e_pad - e_cnt))
        dst = jnp.pad(ei[1], (0, e_pad - e_cnt), constant_values=n)
        return src, dst, e_pad

    src_p, dst_p, e_pad = pad_edges(edge_index, e)
    srcr_p, dstr_p, er_pad = pad_edges(edge_index_ref, e_ref)
    assert e_pad == er_pad
    srcn_p = src_p + n

    z4 = jnp.zeros((NT, HEADS), jnp.float32)
    s1 = _make_s1(e_pad, n)
    ex_hgt, ex_han, s_hgt, s_han = s1(
        src_p, dst_p, srcr_p, dstr_p, qs_p, kt, c_p, asrc, adst_p, cr_p, z4)

    rsh, rsr = pl.pallas_call(
        _tmid_body,
        out_shape=[jax.ShapeDtypeStruct((NT, HEADS), jnp.float32),
                   jax.ShapeDtypeStruct((NT, HEADS), jnp.float32)],
    )(s_hgt, s_han)

    v2f = v2.reshape(2 * n, 128)
    z128 = jnp.zeros((NT, 128), jnp.float32)
    z64 = jnp.zeros((NT, HAN_OUT), jnp.float32)
    s3 = _make_s3(e_pad, n)
    alpha_pad, olo, ohi, orr = s3(
        src_p, srcn_p, dst_p, srcr_p, dstr_p, ex_hgt, ex_han, rsh, rsr,
        v2f, xr, z128, z64)

    emb = pl.pallas_call(
        _t2_body,
        grid=(grid,),
        in_specs=[
            pl.BlockSpec((2, BR, 128), lambda i: (0, i, 0)),
            pl.BlockSpec((2, BR, 128), lambda i: (0, i, 0)),
            pl.BlockSpec((2, BR, HAN_OUT), lambda i: (0, i, 0)),
            pl.BlockSpec((BR, HIDDEN), lambda i: (i, 0)),
            bw((HIDDEN, HIDDEN)), bw((1, HIDDEN)), bw((1, 1)),
            bw((HIDDEN, HIDDEN)), bw((HAN_OUT, HIDDEN)), bw((1, HIDDEN)),
        ],
        out_specs=pl.BlockSpec((BR, HIDDEN), lambda i: (i, 0)),
        out_shape=jax.ShapeDtypeStruct((n, HIDDEN), jnp.float32),
    )(olo[:, :n, :], ohi[:, :n, :], orr[:, :n, :], h, W_out, b_out[None, :],
      skip.reshape(1, 1), W_fin[:HIDDEN], W_fin[HIDDEN:], b_fin[None, :])

    return emb, alpha_pad[:e]


# speedup vs baseline: 11.3285x; 11.3285x over previous
import functools
import math

import jax
import jax.numpy as jnp
from jax import lax
from jax.experimental import pallas as pl
from jax.experimental.pallas import tpu as pltpu
from jax.experimental.pallas import tpu_sc as plsc

HEADS = 4
HP = 8
D_HEAD = 64
HIDDEN = 256
HAN_OUT = 64
HAN_HEADS = 4
HAN_D = 16

NC = 2
NS = 16
LANES = 16
NW = NC * NS

CH = 128
NT = 10240
RPT = NT // NS

BR = 400


def _t1_body(xr_ref, x_ref_, wproj, bproj, as_m, ad_m, wlin, blin, wq, bq, wk, bk,
             wv, bv, bq_rel, bv_rel, qscale, mh,
             h_o, qs_o, kt_o, v2_o, xrh_o, asrc_o, adst_o, nq_o, nk_o):
    xr = xr_ref[...]
    xp = jnp.dot(xr, wproj[...], preferred_element_type=jnp.float32) + bproj[...]
    xrh_o[...] = xp
    asrc_o[...] = jnp.dot(xp, as_m[...], preferred_element_type=jnp.float32)
    adst_o[...] = jnp.dot(xp, ad_m[...], preferred_element_type=jnp.float32)

    x = x_ref_[...]
    h = jnp.maximum(jnp.dot(x, wlin[...], preferred_element_type=jnp.float32) + blin[...], 0.0)
    h_o[...] = h
    q = jnp.dot(h, wq[...], preferred_element_type=jnp.float32) + bq[...]
    k = jnp.dot(h, wk[...], preferred_element_type=jnp.float32) + bk[...]
    v = jnp.dot(h, wv[...], preferred_element_type=jnp.float32) + bv[...]
    kt = jnp.dot(k, bq_rel[...], preferred_element_type=jnp.float32)
    vt = jnp.dot(v, bv_rel[...], preferred_element_type=jnp.float32)
    qs = q * qscale[...]
    qs_o[...] = qs
    kt_o[...] = kt
    v2_o[0] = vt[:, :128]
    v2_o[1] = vt[:, 128:]
    nq_o[...] = jnp.sqrt(jnp.dot(qs * qs, mh[...], preferred_element_type=jnp.float32))
    nk_o[...] = jnp.sqrt(jnp.dot(kt * kt, mh[...], preferred_element_type=jnp.float32))


def _t1b_body(nq_ref, nk_ref, asrc_ref, adst_ref, c_o, cr_o):
    kmax = jnp.max(nk_ref[...], axis=0, keepdims=True)
    c_o[...] = nq_ref[...] * kmax
    amax = jnp.max(asrc_ref[...], axis=0, keepdims=True)
    t = amax + adst_ref[...]
    cr_o[...] = jnp.where(t > 0, t, 0.2 * t)


def _tmid_body(sh_ref, sr_ref, rsh_o, rsr_o):
    rsh_o[...] = 1.0 / (sh_ref[0] + sh_ref[1] + 1e-16)
    rsr_o[...] = 1.0 / (sr_ref[0] + sr_ref[1] + 1e-16)


def _t2_body(lo_ref, hi_ref, orr_ref, h_ref, wout, bout, skip_ref, wf1, wf2, bfin,
             emb_o):
    og = jnp.concatenate([lo_ref[0] + lo_ref[1], hi_ref[0] + hi_ref[1]], axis=1)
    g = og * 0.5 * (1.0 + lax.erf(og * (1.0 / math.sqrt(2.0))))
    out = jnp.dot(g, wout[...], preferred_element_type=jnp.float32) + bout[...]
    beta = jax.nn.sigmoid(skip_ref[0, 0])
    hn = beta * out + (1.0 - beta) * h_ref[...]
    orr = jnp.maximum(orr_ref[0] + orr_ref[1], 0.0)
    emb_o[...] = (jnp.dot(hn, wf1[...], preferred_element_type=jnp.float32)
                  + jnp.dot(orr, wf2[...], preferred_element_type=jnp.float32)
                  + bfin[...])


def _full16(val):
    return jnp.full((LANES,), val, dtype=jnp.int32)


_SC_PARAMS = pltpu.CompilerParams(use_tc_tiling_on_sc=False,
                                  needs_layout_passes=False)


def _make_s1(e_pad, n):
    ew = e_pad // NW
    nchunk = ew // CH
    mesh = plsc.VectorSubcoreMesh(core_axis_name="c", subcore_axis_name="s")

    @functools.partial(
        pl.kernel,
        mesh=mesh,
        compiler_params=_SC_PARAMS,
        out_type=[
            jax.ShapeDtypeStruct((e_pad, HP), jnp.float32),
            jax.ShapeDtypeStruct((e_pad, HP), jnp.float32),
            jax.ShapeDtypeStruct((NC, NT, HP), jnp.float32),
            jax.ShapeDtypeStruct((NC, NT, HP), jnp.float32),
        ],
        scratch_types=[
            pltpu.VMEM((5, CH), jnp.int32),
            pltpu.VMEM((CH, HIDDEN), jnp.float32),
            pltpu.VMEM((CH, HIDDEN), jnp.float32),
            pltpu.VMEM((CH, HP), jnp.float32),
            pltpu.VMEM((CH, HP), jnp.float32),
            pltpu.VMEM((CH, HP), jnp.float32),
            pltpu.VMEM((CH, HP), jnp.float32),
            pltpu.VMEM((CH, HP), jnp.float32),
            pltpu.VMEM((CH, HP), jnp.float32),
            pltpu.VMEM_SHARED((NT, HP), jnp.float32),
            pltpu.VMEM_SHARED((NT, HP), jnp.float32),
            pltpu.SemaphoreType.DMA,
        ],
    )
    def s1(eidx_h, qs_h, kt_h, c_h, asrc_h, adst_h, cr_h, z4_h,
           ex_hgt_h, ex_han_h, s_hgt_h, s_han_h,
           idxb, qrows, krows, crows, asr, adr, crr, exb, exrb,
           stab, stab_r, sem):
        cid = lax.axis_index("c")
        sid = lax.axis_index("s")
        wid = sid * NC + cid
        r0 = sid * RPT
        pltpu.sync_copy(z4_h.at[pl.ds(r0, RPT)], stab.at[pl.ds(r0, RPT)])
        pltpu.sync_copy(z4_h.at[pl.ds(r0, RPT)], stab_r.at[pl.ds(r0, RPT)])
        plsc.subcore_barrier()
        lane = lax.iota(jnp.int32, LANES)
        zf = jnp.zeros((LANES,), jnp.float32)
        for g in range(CH // LANES):
            lids = lane + (g * LANES)
            for h in range(HP):
                plsc.store_scatter(exb, [lids, _full16(h)], zf)
                plsc.store_scatter(exrb, [lids, _full16(h)], zf)
        base_e = wid * ew

        def chunk(ci, carry):
            eb = base_e + ci * CH
            pltpu.sync_copy(eidx_h.at[:, pl.ds(eb, CH)], idxb)
            dv = idxb.at[0]
            sv = idxb.at[1]
            srv = idxb.at[3]
            drv = idxb.at[4]
            cps = [
                pltpu.async_copy(qs_h.at[dv], qrows, sem),
                pltpu.async_copy(kt_h.at[sv], krows, sem),
                pltpu.async_copy(c_h.at[dv], crows, sem),
                pltpu.async_copy(asrc_h.at[srv], asr, sem),
                pltpu.async_copy(adst_h.at[drv], adr, sem),
                pltpu.async_copy(cr_h.at[drv], crr, sem),
            ]
            for cp in cps:
                cp.wait()
            for g in range(CH // LANES):
                lids = lane + (g * LANES)

                def dstep(d, accs):
                    cb = jnp.full((LANES,), d, dtype=jnp.int32)
                    new = []
                    for h in range(HEADS):
                        col = cb + (h * D_HEAD)
                        qv = plsc.load_gather(qrows, [lids, col])
                        kv = plsc.load_gather(krows, [lids, col])
                        new.append(accs[h] + qv * kv)
                    return tuple(new)

                zero = jnp.zeros((LANES,), jnp.float32)
                accs = lax.fori_loop(0, D_HEAD, dstep, (zero, zero, zero, zero))
                for h in range(HEADS):
                    hcol = _full16(h)
                    cv = plsc.load_gather(crows, [lids, hcol])
                    ex = jnp.exp(accs[h] - cv)
                    plsc.store_scatter(exb, [lids, hcol], ex)
                for h in range(HEADS):
                    hcol = _full16(h)
                    a = (plsc.load_gather(asr, [lids, hcol])
                         + plsc.load_gather(adr, [lids, hcol]))
                    a = jnp.where(a > 0, a, 0.2 * a)
                    exr = jnp.exp(a - plsc.load_gather(crr, [lids, hcol]))
                    plsc.store_scatter(exrb, [lids, hcol], exr)
            wps = [
                pltpu.async_copy(exb, ex_hgt_h.at[pl.ds(eb, CH)], sem),
                pltpu.async_copy(exrb, ex_han_h.at[pl.ds(eb, CH)], sem),
            ]
            pltpu.sync_copy(exb, stab.at[dv], add=True)
            pltpu.sync_copy(exrb, stab_r.at[drv], add=True)
            for wp in wps:
                wp.wait()
            return carry

        lax.fori_loop(0, nchunk, chunk, 0)
        plsc.subcore_barrier()
        pltpu.sync_copy(stab.at[pl.ds(r0, RPT)], s_hgt_h.at[cid, pl.ds(r0, RPT)])
        pltpu.sync_copy(stab_r.at[pl.ds(r0, RPT)], s_han_h.at[cid, pl.ds(r0, RPT)])

    return s1


def _make_s3(e_pad, n):
    ew = e_pad // NW
    nchunk = ew // CH
    mesh = plsc.VectorSubcoreMesh(core_axis_name="c", subcore_axis_name="s")
    HALF = 128

    @functools.partial(
        pl.kernel,
        mesh=mesh,
        compiler_params=_SC_PARAMS,
        out_type=[
            jax.ShapeDtypeStruct((e_pad, HP), jnp.float32),
            jax.ShapeDtypeStruct((NC, NT, HALF), jnp.float32),
            jax.ShapeDtypeStruct((NC, NT, HALF), jnp.float32),
        ],
        scratch_types=[
            pltpu.VMEM((3, CH), jnp.int32),
            pltpu.VMEM((CH, HP), jnp.float32),
            pltpu.VMEM((CH, HP), jnp.float32),
            pltpu.VMEM((CH, HP), jnp.float32),
            pltpu.VMEM((CH, HALF), jnp.float32),
            pltpu.VMEM((CH, HALF), jnp.float32),
            pltpu.VMEM_SHARED((NT, HALF), jnp.float32),
            pltpu.SemaphoreType.DMA,
        ],
    )
    def s3(eidx_h, ex_hgt_h, rsh_h, v2_h, z128_h,
           al_h, olo_h, ohi_h,
           idxb, exb, rsb, alb, vrows, msgb, otab, sem):
        cid = lax.axis_index("c")
        sid = lax.axis_index("s")
        wid = sid * NC + cid
        r0 = sid * RPT
        lane = lax.iota(jnp.int32, LANES)
        base_e = wid * ew

        pltpu.sync_copy(z128_h.at[pl.ds(r0, RPT)], otab.at[pl.ds(r0, RPT)])
        plsc.subcore_barrier()

        def do_pass(half):
            h0, h1 = (0, 1) if half == 0 else (2, 3)
            srow = 1 if half == 0 else 2

            def chunk(ci, carry):
                eb = base_e + ci * CH
                pltpu.sync_copy(eidx_h.at[pl.ds(0, 3), pl.ds(eb, CH)], idxb)
                dv = idxb.at[0]
                sv = idxb.at[srow]
                cps = [
                    pltpu.async_copy(ex_hgt_h.at[pl.ds(eb, CH)], exb, sem),
                    pltpu.async_copy(rsh_h.at[dv], rsb, sem),
                    pltpu.async_copy(v2_h.at[sv], vrows, sem),
                ]
                for cp in cps:
                    cp.wait()
                for g in range(CH // LANES):
                    lids = lane + (g * LANES)
                    for h in range(HEADS):
                        hcol = _full16(h)
                        al = (plsc.load_gather(exb, [lids, hcol])
                              * plsc.load_gather(rsb, [lids, hcol]))
                        plsc.store_scatter(alb, [lids, hcol], al)

                def vbody(ei, c2):
                    av0 = plsc.load_gather(alb, [_full16(ei), _full16(h0)])
                    av1 = plsc.load_gather(alb, [_full16(ei), _full16(h1)])
                    for c8 in range(HALF // LANES):
                        av = av0 if c8 < 4 else av1
                        v16 = vrows[ei, pl.ds(c8 * LANES, LANES)]
                        msgb[ei, pl.ds(c8 * LANES, LANES)] = v16 * av
                    return c2

                lax.fori_loop(0, CH, vbody, 0)
                if half == 0:
                    wp = pltpu.async_copy(alb, al_h.at[pl.ds(eb, CH)], sem)
                    pltpu.sync_copy(msgb, otab.at[dv], add=True)
                    wp.wait()
                else:
                    pltpu.sync_copy(msgb, otab.at[dv], add=True)
                return carry

            lax.fori_loop(0, nchunk, chunk, 0)

        do_pass(0)
        plsc.subcore_barrier()
        pltpu.sync_copy(otab.at[pl.ds(r0, RPT)], olo_h.at[cid, pl.ds(r0, RPT)])
        plsc.subcore_barrier()
        pltpu.sync_copy(z128_h.at[pl.ds(r0, RPT)], otab.at[pl.ds(r0, RPT)])
        plsc.subcore_barrier()
        do_pass(1)
        plsc.subcore_barrier()
        pltpu.sync_copy(otab.at[pl.ds(r0, RPT)], ohi_h.at[cid, pl.ds(r0, RPT)])

    return s3


def _make_s3b(e_pad, n):
    ew = e_pad // NW
    nchunk = ew // CH
    mesh = plsc.VectorSubcoreMesh(core_axis_name="c", subcore_axis_name="s")

    @functools.partial(
        pl.kernel,
        mesh=mesh,
        compiler_params=_SC_PARAMS,
        out_type=[
            jax.ShapeDtypeStruct((NC, NT, HAN_OUT), jnp.float32),
        ],
        scratch_types=[
            pltpu.VMEM((2, CH), jnp.int32),
            pltpu.VMEM((CH, HP), jnp.float32),
            pltpu.VMEM((CH, HP), jnp.float32),
            pltpu.VMEM((CH, HP), jnp.float32),
            pltpu.VMEM((CH, HAN_OUT), jnp.float32),
            pltpu.VMEM((CH, HAN_OUT), jnp.float32),
            pltpu.VMEM_SHARED((NT, HAN_OUT), jnp.float32),
            pltpu.SemaphoreType.DMA,
        ],
    )
    def s3b(eidx_h, ex_han_h, rsr_h, xr_h, z64_h,
            or_h,
            idxb, exrb, rsrb, alrb, xrows, msgrb, ortab, sem):
        cid = lax.axis_index("c")
        sid = lax.axis_index("s")
        wid = sid * NC + cid
        r0 = sid * RPT
        lane = lax.iota(jnp.int32, LANES)
        base_e = wid * ew

        pltpu.sync_copy(z64_h.at[pl.ds(r0, RPT)], ortab.at[pl.ds(r0, RPT)])
        plsc.subcore_barrier()

        def chunk(ci, carry):
            eb = base_e + ci * CH
            pltpu.sync_copy(eidx_h.at[pl.ds(3, 2), pl.ds(eb, CH)], idxb)
            srv = idxb.at[0]
            drv = idxb.at[1]
            cps = [
                pltpu.async_copy(ex_han_h.at[pl.ds(eb, CH)], exrb, sem),
                pltpu.async_copy(rsr_h.at[drv], rsrb, sem),
                pltpu.async_copy(xr_h.at[srv], xrows, sem),
            ]
            for cp in cps:
                cp.wait()
            for g in range(CH // LANES):
                lids = lane + (g * LANES)
                for h in range(HEADS):
                    hcol = _full16(h)
                    al = (plsc.load_gather(exrb, [lids, hcol])
                          * plsc.load_gather(rsrb, [lids, hcol]))
                    plsc.store_scatter(alrb, [lids, hcol], al)

            def rbody(ei, c2):
                for h in range(HAN_HEADS):
                    av = plsc.load_gather(alrb, [_full16(ei), _full16(h)])
                    x16 = xrows[ei, pl.ds(h * HAN_D, LANES)]
                    msgrb[ei, pl.ds(h * HAN_D, LANES)] = x16 * av
                return c2

            lax.fori_loop(0, CH, rbody, 0)
            pltpu.sync_copy(msgrb, ortab.at[drv], add=True)
            return carry

        lax.fori_loop(0, nchunk, chunk, 0)
        plsc.subcore_barrier()
        pltpu.sync_copy(ortab.at[pl.ds(r0, RPT)], or_h.at[cid, pl.ds(r0, RPT)])

    return s3b


def kernel(x, edge_index, x_ref, edge_index_ref, W_lin, b_lin, W_kqv, b_kqv,
           W_krel, W_vrel, p_rel, W_out, b_out, skip, W_proj, b_proj, att_src,
           att_dst, W_klin, b_klin, q_sem, W_fin, b_fin):
    n = x.shape[0]
    e = edge_index.shape[1]
    e_ref = edge_index_ref.shape[1]
    d_in = x.shape[1]

    wk = W_kqv[:, :HIDDEN]
    wq = W_kqv[:, HIDDEN:2 * HIDDEN]
    wv = W_kqv[:, 2 * HIDDEN:]
    bk = b_kqv[:HIDDEN]
    bq = b_kqv[HIDDEN:2 * HIDDEN]
    bv = b_kqv[2 * HIDDEN:]
    bq_rel = jax.scipy.linalg.block_diag(*[W_krel[h] for h in range(HEADS)])
    bv_rel = jax.scipy.linalg.block_diag(*[W_vrel[h] for h in range(HEADS)])
    qscale = jnp.repeat(p_rel / math.sqrt(D_HEAD), D_HEAD)[None, :]
    mh = jax.scipy.linalg.block_diag(*[jnp.ones((D_HEAD, 1), jnp.float32)
                                       for _ in range(HEADS)])
    mh = jnp.pad(mh, ((0, 0), (0, HP - HEADS)))
    as_m = jax.scipy.linalg.block_diag(*[att_src[h][:, None]
                                         for h in range(HAN_HEADS)])
    as_m = jnp.pad(as_m, ((0, 0), (0, HP - HAN_HEADS)))
    ad_m = jax.scipy.linalg.block_diag(*[att_dst[h][:, None]
                                         for h in range(HAN_HEADS)])
    ad_m = jnp.pad(ad_m, ((0, 0), (0, HP - HAN_HEADS)))

    grid = n // BR
    bw = lambda shp: pl.BlockSpec(shp, lambda *i: tuple(0 for _ in shp))
    t1 = pl.pallas_call(
        _t1_body,
        grid=(grid,),
        in_specs=[
            pl.BlockSpec((BR, d_in), lambda i: (i, 0)),
            pl.BlockSpec((BR, d_in), lambda i: (i, 0)),
            bw((d_in, HAN_OUT)), bw((1, HAN_OUT)),
            bw((HAN_OUT, HP)), bw((HAN_OUT, HP)),
            bw((d_in, HIDDEN)), bw((1, HIDDEN)),
            bw((HIDDEN, HIDDEN)), bw((1, HIDDEN)),
            bw((HIDDEN, HIDDEN)), bw((1, HIDDEN)),
            bw((HIDDEN, HIDDEN)), bw((1, HIDDEN)),
            bw((HIDDEN, HIDDEN)), bw((HIDDEN, HIDDEN)),
            bw((1, HIDDEN)), bw((HIDDEN, HP)),
        ],
        out_specs=[
            pl.BlockSpec((BR, HIDDEN), lambda i: (i, 0)),
            pl.BlockSpec((BR, HIDDEN), lambda i: (i, 0)),
            pl.BlockSpec((BR, HIDDEN), lambda i: (i, 0)),
            pl.BlockSpec((2, BR, 128), lambda i: (0, i, 0)),
            pl.BlockSpec((BR, HAN_OUT), lambda i: (i, 0)),
            pl.BlockSpec((BR, HP), lambda i: (i, 0)),
            pl.BlockSpec((BR, HP), lambda i: (i, 0)),
            pl.BlockSpec((BR, HP), lambda i: (i, 0)),
            pl.BlockSpec((BR, HP), lambda i: (i, 0)),
        ],
        out_shape=[
            jax.ShapeDtypeStruct((n, HIDDEN), jnp.float32),
            jax.ShapeDtypeStruct((n, HIDDEN), jnp.float32),
            jax.ShapeDtypeStruct((n, HIDDEN), jnp.float32),
            jax.ShapeDtypeStruct((2, n, 128), jnp.float32),
            jax.ShapeDtypeStruct((n, HAN_OUT), jnp.float32),
            jax.ShapeDtypeStruct((n, HP), jnp.float32),
            jax.ShapeDtypeStruct((n, HP), jnp.float32),
            jax.ShapeDtypeStruct((n, HP), jnp.float32),
            jax.ShapeDtypeStruct((n, HP), jnp.float32),
        ],
    )(x_ref, x, W_proj, b_proj[None, :], as_m, ad_m, W_lin, b_lin[None, :],
      wq, bq[None, :], wk, bk[None, :], wv, bv[None, :], bq_rel, bv_rel,
      qscale, mh)
    h, qs, kt, v2, xr, asrc, adst, nq, nk = t1

    c, cr = pl.pallas_call(
        _t1b_body,
        out_shape=[jax.ShapeDtypeStruct((n, HP), jnp.float32),
                   jax.ShapeDtypeStruct((n, HP), jnp.float32)],
    )(nq, nk, asrc, adst)

    pad8 = lambda a: jnp.pad(a, ((0, 8), (0, 0)))
    qs_p = pad8(qs)
    c_p = pad8(c)
    adst_p = pad8(adst)
    cr_p = pad8(cr)

    def pad_edges(ei, e_cnt):
        e_pad = ((e_cnt + NW * CH - 1) // (NW * CH)) * (NW * CH)
        src = jnp.pad(ei[0], (0, e_pad - e_cnt))
        dst = jnp.pad(ei[1], (0, e_pad - e_cnt), constant_values=n)
        return src, dst, e_pad

    src_p, dst_p, e_pad = pad_edges(edge_index, e)
    srcr_p, dstr_p, er_pad = pad_edges(edge_index_ref, e_ref)
    assert e_pad == er_pad
    eidx = jnp.stack([dst_p, src_p, src_p + n, srcr_p, dstr_p])

    z4 = jnp.zeros((NT, HP), jnp.float32)
    s1 = _make_s1(e_pad, n)
    ex_hgt, ex_han, s_hgt, s_han = s1(
        eidx, qs_p, kt, c_p, asrc, adst_p, cr_p, z4)

    rsh, rsr = pl.pallas_call(
        _tmid_body,
        out_shape=[jax.ShapeDtypeStruct((NT, HP), jnp.float32),
                   jax.ShapeDtypeStruct((NT, HP), jnp.float32)],
    )(s_hgt, s_han)

    v2f = v2.reshape(2 * n, 128)
    z128 = jnp.zeros((NT, 128), jnp.float32)
    z64 = jnp.zeros((NT, HAN_OUT), jnp.float32)
    s3 = _make_s3(e_pad, n)
    alpha_pad, olo, ohi = s3(eidx, ex_hgt, rsh, v2f, z128)
    s3b = _make_s3b(e_pad, n)
    (orr,) = s3b(eidx, ex_han, rsr, xr, z64)

    emb = pl.pallas_call(
        _t2_body,
        grid=(grid,),
        in_specs=[
            pl.BlockSpec((2, BR, 128), lambda i: (0, i, 0)),
            pl.BlockSpec((2, BR, 128), lambda i: (0, i, 0)),
            pl.BlockSpec((2, BR, HAN_OUT), lambda i: (0, i, 0)),
            pl.BlockSpec((BR, HIDDEN), lambda i: (i, 0)),
            bw((HIDDEN, HIDDEN)), bw((1, HIDDEN)), bw((1, 1)),
            bw((HIDDEN, HIDDEN)), bw((HAN_OUT, HIDDEN)), bw((1, HIDDEN)),
        ],
        out_specs=pl.BlockSpec((BR, HIDDEN), lambda i: (i, 0)),
        out_shape=jax.ShapeDtypeStruct((n, HIDDEN), jnp.float32),
    )(olo[:, :n, :], ohi[:, :n, :], orr[:, :n, :], h, W_out, b_out[None, :],
      skip.reshape(1, 1), W_fin[:HIDDEN], W_fin[HIDDEN:], b_fin[None, :])

    return emb, alpha_pad[:e, :HEADS]
